# baseline (speedup 1.0000x reference)
import functools

import jax
import jax.numpy as jnp
from jax import lax
from jax.experimental import pallas as pl
from jax.experimental.pallas import tpu as pltpu
from jax.experimental.pallas import tpu_sc as plsc

ALPHA_C = 1000.0
L = 16


def _rsqrt16(x):
    i = lax.bitcast_convert_type(x, jnp.int32)
    i = jnp.int32(0x5F3759DF) - lax.shift_right_arithmetic(i, 1)
    r = lax.bitcast_convert_type(i, jnp.float32)
    for _ in range(3):
        r = r * (1.5 - 0.5 * x * r * r)
    return r


def _make_sc_kernel(n_nodes_pad, n_edges, n_graphs, chunk):
    NC, NS = 2, 16
    NW = NC * NS
    per_w = n_edges // NW
    n_chunks = per_w // chunk
    assert n_chunks % 2 == 0
    nodes_per_tile = n_nodes_pad // NS
    vecs_per_chunk = chunk // L

    mesh = plsc.VectorSubcoreMesh(core_axis_name="c", subcore_axis_name="s")

    def edge_buf():
        return [
            pltpu.VMEM((chunk,), jnp.int32),
            pltpu.VMEM((chunk,), jnp.int32),
            pltpu.VMEM((chunk * 2,), jnp.float32),
            pltpu.VMEM((chunk,), jnp.float32),
            pltpu.VMEM((chunk,), jnp.float32),
            pltpu.VMEM((chunk,), jnp.float32),
            pltpu.VMEM((chunk,), jnp.float32),
            pltpu.VMEM((chunk,), jnp.float32),
            pltpu.VMEM((chunk,), jnp.float32),
            pltpu.VMEM((chunk,), jnp.int32),
        ]

    @functools.partial(
        pl.kernel,
        out_type=jax.ShapeDtypeStruct((NC * 64,), jnp.float32),
        mesh=mesh,
        compiler_params=pltpu.CompilerParams(
            needs_layout_passes=False, use_tc_tiling_on_sc=False),
        scratch_types=[
            pltpu.VMEM_SHARED((n_nodes_pad,), jnp.float32),
            pltpu.VMEM_SHARED((n_nodes_pad,), jnp.float32),
            pltpu.VMEM_SHARED((n_nodes_pad,), jnp.float32),
            pltpu.VMEM_SHARED((n_nodes_pad,), jnp.int32),
            pltpu.VMEM_SHARED((NS, n_graphs * L), jnp.float32),
            pltpu.VMEM((nodes_per_tile,), jnp.float32),
            pltpu.VMEM((nodes_per_tile,), jnp.int32),
            pltpu.VMEM((64,), jnp.int32),
            edge_buf(),
            edge_buf(),
            pltpu.VMEM((n_graphs * L,), jnp.float32),
            pltpu.VMEM((NS, n_graphs * L), jnp.float32),
            pltpu.VMEM((64,), jnp.float32),
            pltpu.SemaphoreType.DMA,
            pltpu.SemaphoreType.DMA,
            pltpu.SemaphoreType.DMA,
            pltpu.SemaphoreType.DMA,
        ],
    )
    def sc_kernel(px_hbm, py_hbm, pz_hbm, src_hbm, dst_hbm, attr_hbm,
                  ptr_hbm, out_hbm,
                  px_sh, py_sh, pz_sh, g_sh, acc_sh,
                  stage_v, gstage_v, ptr_v, bufs_a, bufs_b,
                  acc_v, accall_v, out_v,
                  semL_a, semL_b, semG_a, semG_b):
        cid = lax.axis_index("c")
        sid = lax.axis_index("s")
        wid = cid * NS + sid

        node_lo = sid * nodes_per_tile
        for src_ref, dst_ref in ((px_hbm, px_sh), (py_hbm, py_sh),
                                 (pz_hbm, pz_sh)):
            pltpu.sync_copy(src_ref.at[pl.ds(node_lo, nodes_per_tile)], stage_v)
            pltpu.sync_copy(stage_v, dst_ref.at[pl.ds(node_lo, nodes_per_tile)])

        pltpu.sync_copy(ptr_hbm, ptr_v)
        ptr_vecs = [ptr_v[pl.ds(k * L, L)] for k in range(64 // L)]
        bounds = [ptr_vecs[j // L][j % L] for j in range(1, n_graphs)]
        lane = lax.iota(jnp.int32, L)

        def g_body(k, _):
            n = node_lo + k * L + lane
            cnt = jnp.zeros((L,), jnp.int32)
            for b in bounds:
                cnt = cnt + jnp.where(n >= b, 1, 0).astype(jnp.int32)
            gstage_v[pl.ds(k * L, L)] = cnt
            return 0

        lax.fori_loop(0, nodes_per_tile // L, g_body, 0)
        pltpu.sync_copy(gstage_v, g_sh.at[pl.ds(node_lo, nodes_per_tile)])

        zero16 = jnp.zeros((L,), jnp.float32)

        def z_body(i, _):
            acc_v[pl.ds(i * L, L)] = zero16
            return 0

        lax.fori_loop(0, n_graphs, z_body, 0)

        plsc.subcore_barrier()

        edge_base = wid * per_w

        def linear_descs(bufs, sem, lo, make):
            f = pltpu.make_async_copy if make else pltpu.async_copy
            return [
                f(src_hbm.at[pl.ds(lo, chunk)], bufs[0], sem),
                f(dst_hbm.at[pl.ds(lo, chunk)], bufs[1], sem),
                f(attr_hbm.at[pl.ds(lo * 2, chunk * 2)], bufs[2], sem),
            ]

        def gather_descs(bufs, sem, make):
            f = pltpu.make_async_copy if make else pltpu.async_copy
            return [
                f(px_sh.at[bufs[0]], bufs[3], sem),
                f(py_sh.at[bufs[0]], bufs[4], sem),
                f(pz_sh.at[bufs[0]], bufs[5], sem),
                f(px_sh.at[bufs[1]], bufs[6], sem),
                f(py_sh.at[bufs[1]], bufs[7], sem),
                f(pz_sh.at[bufs[1]], bufs[8], sem),
                f(g_sh.at[bufs[0]], bufs[9], sem),
            ]

        def fire_chunk(bufs, semL, semG, lo):
            linear_descs(bufs, semL, lo, False)
            for d in linear_descs(bufs, semL, lo, True):
                d.wait()
            gather_descs(bufs, semG, False)

        def drain_gathers(bufs, semG):
            for d in gather_descs(bufs, semG, True):
                d.wait()

        def compute(bufs):
            attr_v = bufs[2]
            sx_v, sy_v, sz_v = bufs[3], bufs[4], bufs[5]
            dx_v, dy_v, dz_v = bufs[6], bufs[7], bufs[8]
            gv_v = bufs[9]

            def vec_body(k, _):
                o = k * L
                ddx = sx_v[pl.ds(o, L)] - dx_v[pl.ds(o, L)]
                ddy = sy_v[pl.ds(o, L)] - dy_v[pl.ds(o, L)]
                ddz = sz_v[pl.ds(o, L)] - dz_v[pl.ds(o, L)]
                d2 = ddx * ddx + ddy * ddy + ddz * ddz + 1e-12
                dist = d2 * _rsqrt16(d2)
                eidx2 = (o + lane) * 2
                r0 = plsc.load_gather(attr_v, [eidx2])
                w = plsc.load_gather(attr_v, [eidx2 + 1])
                diff = dist - r0
                e = (ALPHA_C * w) * (diff * diff)
                g = gv_v[pl.ds(o, L)]
                plsc.addupdate_scatter(acc_v, [g * L + lane], e)
                return 0

            lax.fori_loop(0, vecs_per_chunk, vec_body, 0)

        fire_chunk(bufs_a, semL_a, semG_a, pl.multiple_of(edge_base, 8))

        nb = n_chunks // 2

        def pipe_body(j, _):
            lo_b = pl.multiple_of(edge_base + (2 * j + 1) * chunk, 8)
            fire_chunk(bufs_b, semL_b, semG_b, lo_b)
            drain_gathers(bufs_a, semG_a)
            compute(bufs_a)

            @pl.when(j < nb - 1)
            def _():
                lo_a = pl.multiple_of(edge_base + (2 * j + 2) * chunk, 8)
                fire_chunk(bufs_a, semL_a, semG_a, lo_a)

            drain_gathers(bufs_b, semG_b)
            compute(bufs_b)
            return 0

        lax.fori_loop(0, nb, pipe_body, 0)

        pltpu.sync_copy(acc_v, acc_sh.at[sid])
        plsc.subcore_barrier()

        @pl.when(sid == 0)
        def _():
            pltpu.sync_copy(acc_sh, accall_v)
            for k in range(64 // L):
                row = zero16
                for j in range(L):
                    gi = k * L + j
                    if gi >= n_graphs:
                        break
                    tot = accall_v[0, pl.ds(gi * L, L)]
                    for t in range(1, NS):
                        tot = tot + accall_v[t, pl.ds(gi * L, L)]
                    row = jnp.where(lane == j, jnp.sum(tot), row)
                out_v[pl.ds(k * L, L)] = row
            pltpu.sync_copy(out_v, out_hbm.at[pl.ds(cid * 64, 64)])

    return sc_kernel


def kernel(positions, edge_attrs, edge_index, ptr):
    n_nodes = positions.shape[0]
    n_edges = edge_index.shape[1]
    n_graphs = ptr.shape[0] - 1

    n_nodes_pad = ((n_nodes + 127) // 128) * 128
    pad = n_nodes_pad - n_nodes
    px = jnp.pad(positions[:, 0], (0, pad))
    py = jnp.pad(positions[:, 1], (0, pad))
    pz = jnp.pad(positions[:, 2], (0, pad))
    src = edge_index[0]
    dst = edge_index[1]
    ptr64 = jnp.pad(ptr, (0, 64 - ptr.shape[0]))

    sc = _make_sc_kernel(n_nodes_pad, n_edges, n_graphs, chunk=2000)
    out2 = sc(px, py, pz, src, dst, edge_attrs.reshape(-1), ptr64)
    return (out2[:64] + out2[64:])[:n_graphs]
